# Initial kernel scaffold; baseline (speedup 1.0000x reference)
import functools

import jax
import jax.numpy as jnp
from jax import lax
from jax.experimental import pallas as pl
from jax.experimental.pallas import tpu as pltpu
from jax.experimental.pallas import tpu_sc as plsc

S = 2048
D = 1024
F = 2048
V = 8192
E = 8
CAP = 640
A = 2 * S
XE_ROWS = E * CAP + 8
EPS = 1e-5
_LANES = 128
_NEG = jnp.float32(-1e30)


def _sc_mesh():
    return plsc.VectorSubcoreMesh(core_axis_name="core", subcore_axis_name="subcore")


def _sc_gather_rows(table, idx, window=32):
    m = idx.shape[0]
    d = table.shape[1]

    @functools.partial(
        pl.kernel,
        out_type=jax.ShapeDtypeStruct((m, d), table.dtype),
        mesh=_sc_mesh(),
    )
    def k(tab_hbm, idx_hbm, out_hbm):
        def body(i_vmem, o_vmem):
            pltpu.sync_copy(tab_hbm.at[i_vmem.at[0]], o_vmem)

        pltpu.emit_pipeline(
            body,
            grid=(m // window,),
            in_specs=[pl.BlockSpec((1, window), lambda i: (0, i))],
            out_specs=[pl.BlockSpec((window, d), lambda i: (i, 0))],
            core_axis_name=("core", "subcore"),
            dimension_semantics=(pltpu.PARALLEL,),
        )(idx_hbm, out_hbm)

    return k(table, idx.reshape(1, m))


def _sc_dispatch_scatter(x, slot_flat, window=32):
    @functools.partial(
        pl.kernel,
        out_type=jax.ShapeDtypeStruct((XE_ROWS, D), x.dtype),
        mesh=_sc_mesh(),
    )
    def k(x_hbm, slot_hbm, xe_hbm):
        def body(x_vmem, i_vmem):
            pltpu.sync_copy(x_vmem, xe_hbm.at[i_vmem.at[0]])

        pltpu.emit_pipeline(
            body,
            grid=(A // window,),
            in_specs=[
                pl.BlockSpec((window, D), lambda i: (i % (S // window), 0)),
                pl.BlockSpec((1, window), lambda i: (0, i)),
            ],
            out_specs=[],
            core_axis_name=("core", "subcore"),
            dimension_semantics=(pltpu.PARALLEL,),
        )(x_hbm, slot_hbm)

    return k(x, slot_flat.reshape(1, A))


def _router_dispatch_body(x_ref, wr_ref, ss_ref, cs_ref, cw_ref):
    xb = x_ref[...].astype(jnp.bfloat16)
    wr = wr_ref[...].astype(jnp.bfloat16)
    logits = jnp.dot(xb, wr, preferred_element_type=jnp.float32)
    lane = lax.broadcasted_iota(jnp.int32, (S, _LANES), 1)
    logits = jnp.where(lane < E, logits, _NEG)

    m1 = jnp.max(logits, axis=1, keepdims=True)
    a1 = jnp.min(jnp.where(logits == m1, lane, _LANES), axis=1, keepdims=True)
    l2 = jnp.where(lane == a1, _NEG, logits)
    m2 = jnp.max(l2, axis=1, keepdims=True)
    a2 = jnp.min(jnp.where(l2 == m2, lane, _LANES), axis=1, keepdims=True)
    t_ = jnp.exp(m2 - m1)
    w1 = 1.0 / (1.0 + t_)
    w2 = t_ / (1.0 + t_)

    sel_mask = (lane == a1) | (lane == a2)
    mf = jnp.where(sel_mask, 1.0, 0.0)

    r_i = lax.broadcasted_iota(jnp.int32, (_LANES, _LANES), 0)
    c_i = lax.broadcasted_iota(jnp.int32, (_LANES, _LANES), 1)
    tl = (c_i <= r_i).astype(jnp.bfloat16)
    off = jnp.zeros((1, _LANES), jnp.float32)
    blocks = []
    for b in range(S // _LANES):
        mb = mf[b * _LANES:(b + 1) * _LANES, :]
        incl = jnp.dot(tl, mb.astype(jnp.bfloat16), preferred_element_type=jnp.float32)
        blocks.append(incl - mb + off)
        off = off + incl[_LANES - 1:_LANES, :]
    c = jnp.concatenate(blocks, axis=0).astype(jnp.int32)

    slot = lane * CAP + c
    kept = c < CAP
    scat = jnp.where(kept, slot, E * CAP)
    comb = jnp.where(kept, slot, lane * CAP)

    def sel(arr, a):
        return jnp.sum(jnp.where(lane == a, arr, 0), axis=1, keepdims=True)

    ss_ref[...] = jnp.concatenate([sel(scat, a1), sel(scat, a2)], axis=1)
    cs_ref[...] = jnp.concatenate([sel(comb, a1), sel(comb, a2)], axis=1)
    k1 = jnp.where(sel(jnp.where(kept, 1, 0), a1) > 0, w1, 0.0)
    k2 = jnp.where(sel(jnp.where(kept, 1, 0), a2) > 0, w2, 0.0)
    cw_ref[...] = jnp.concatenate([k1, k2], axis=1)


def _router_dispatch(x, wr_pad):
    return pl.pallas_call(
        _router_dispatch_body,
        out_shape=[
            jax.ShapeDtypeStruct((S, 2), jnp.int32),
            jax.ShapeDtypeStruct((S, 2), jnp.int32),
            jax.ShapeDtypeStruct((S, 2), jnp.float32),
        ],
    )(x, wr_pad)


def _ffn_body(xe_ref, w1_ref, b1_ref, w2_ref, b2_ref, he_ref):
    xb = xe_ref[...].astype(jnp.bfloat16)
    h = jnp.dot(xb, w1_ref[0], preferred_element_type=jnp.float32)
    h = jax.nn.gelu(h + b1_ref[0], approximate=True)
    o = jnp.dot(h.astype(jnp.bfloat16), w2_ref[0], preferred_element_type=jnp.float32)
    he_ref[...] = o + b2_ref[0]


def _ffn(xe, w1b, b1r, w2b, b2r):
    return pl.pallas_call(
        _ffn_body,
        grid=(E,),
        in_specs=[
            pl.BlockSpec((CAP, D), lambda e: (e, 0)),
            pl.BlockSpec((1, D, F), lambda e: (e, 0, 0)),
            pl.BlockSpec((1, 1, F), lambda e: (e, 0, 0)),
            pl.BlockSpec((1, F, D), lambda e: (e, 0, 0)),
            pl.BlockSpec((1, 1, D), lambda e: (e, 0, 0)),
        ],
        out_specs=pl.BlockSpec((CAP, D), lambda e: (e, 0)),
        out_shape=jax.ShapeDtypeStruct((E * CAP, D), jnp.float32),
    )(xe, w1b, b1r, w2b, b2r)


def _lm_body(hab_ref, cwa_ref, cwb_ref, g_ref, b_ref, wlm_ref, out_ref, hn_ref):
    v = pl.program_id(0)

    @pl.when(v == 0)
    def _():
        comb = cwa_ref[...] * hab_ref[0:S, :] + cwb_ref[...] * hab_ref[S:A, :]
        mu = jnp.mean(comb, axis=1, keepdims=True)
        dd = comb - mu
        var = jnp.mean(dd * dd, axis=1, keepdims=True)
        hn = dd / jnp.sqrt(var + EPS) * g_ref[...] + b_ref[...]
        hn_ref[...] = hn.astype(jnp.bfloat16)

    out_ref[...] = jnp.dot(hn_ref[...], wlm_ref[...], preferred_element_type=jnp.float32)


def _lm_head(hab, cwa, cwb, g, b, wlmb, vblk=1024):
    return pl.pallas_call(
        _lm_body,
        grid=(V // vblk,),
        in_specs=[
            pl.BlockSpec((A, D), lambda v: (0, 0)),
            pl.BlockSpec((S, 1), lambda v: (0, 0)),
            pl.BlockSpec((S, 1), lambda v: (0, 0)),
            pl.BlockSpec((1, D), lambda v: (0, 0)),
            pl.BlockSpec((1, D), lambda v: (0, 0)),
            pl.BlockSpec((D, vblk), lambda v: (0, v)),
        ],
        out_specs=pl.BlockSpec((S, vblk), lambda v: (0, v)),
        out_shape=jax.ShapeDtypeStruct((S, V), jnp.float32),
        scratch_shapes=[pltpu.VMEM((S, D), jnp.bfloat16)],
    )(hab, cwa, cwb, g, b, wlmb)


def kernel(input_ids, embed, w_router, w1, b1, w2, b2, ln_scale, ln_bias, w_lm):
    ids = input_ids.reshape(S).astype(jnp.int32)
    x = _sc_gather_rows(embed, ids)

    wr_pad = jnp.zeros((D, _LANES), jnp.float32).at[:, :E].set(w_router)
    ss, cs, cw = _router_dispatch(x, wr_pad)

    xe = _sc_dispatch_scatter(x, ss.T.reshape(A))
    he = _ffn(
        xe[:E * CAP],
        w1.astype(jnp.bfloat16),
        b1.reshape(E, 1, F),
        w2.astype(jnp.bfloat16),
        b2.reshape(E, 1, D),
    )

    hab = _sc_gather_rows(he, cs.T.reshape(A))
    logits = _lm_head(
        hab,
        cw[:, 0:1],
        cw[:, 1:2],
        ln_scale.reshape(1, D),
        ln_bias.reshape(1, D),
        w_lm.astype(jnp.bfloat16),
    )
    return logits.reshape(1, S, V)


# speedup vs baseline: 1.3277x; 1.3277x over previous
import functools

import jax
import jax.numpy as jnp
from jax import lax
from jax.experimental import pallas as pl
from jax.experimental.pallas import tpu as pltpu
from jax.experimental.pallas import tpu_sc as plsc

S = 2048
D = 1024
F = 2048
V = 8192
E = 8
CAP = 640
A = 2 * S
XE_ROWS = (E + 1) * CAP
EPS = 1e-5
_LANES = 128
_NEG = -1.0e30


def _sc_mesh():
    return plsc.VectorSubcoreMesh(core_axis_name="core", subcore_axis_name="subcore")


_SEG = 8
_WIN = 128


def _sc_gather_rows(table_seg, idx_seg):
    m8 = idx_seg.shape[0]

    @functools.partial(
        pl.kernel,
        out_type=jax.ShapeDtypeStruct((m8, _WIN), table_seg.dtype),
        mesh=_sc_mesh(),
    )
    def k(tab_hbm, idx_hbm, out_hbm):
        def body(i_vmem, o_vmem):
            pltpu.sync_copy(tab_hbm.at[i_vmem.at[0]], o_vmem)

        pltpu.emit_pipeline(
            body,
            grid=(m8 // _WIN,),
            in_specs=[pl.BlockSpec((1, _WIN), lambda i: (0, i))],
            out_specs=[pl.BlockSpec((_WIN, _WIN), lambda i: (i, 0))],
            core_axis_name=("core", "subcore"),
            dimension_semantics=(pltpu.PARALLEL,),
        )(idx_hbm, out_hbm)

    return k(table_seg, idx_seg.reshape(1, m8))


def _sc_dispatch_scatter(x_seg, seg_slot):
    @functools.partial(
        pl.kernel,
        out_type=jax.ShapeDtypeStruct((XE_ROWS * _SEG, _WIN), x_seg.dtype),
        mesh=_sc_mesh(),
    )
    def k(x_hbm, slot_hbm, xe_hbm):
        def body(x_vmem, i_vmem):
            pltpu.sync_copy(x_vmem, xe_hbm.at[i_vmem.at[0]])

        nsrc = (S * _SEG) // _WIN
        pltpu.emit_pipeline(
            body,
            grid=((A * _SEG) // _WIN,),
            in_specs=[
                pl.BlockSpec((_WIN, _WIN), lambda i: (i % nsrc, 0)),
                pl.BlockSpec((1, _WIN), lambda i: (0, i)),
            ],
            out_specs=[],
            core_axis_name=("core", "subcore"),
            dimension_semantics=(pltpu.PARALLEL,),
        )(x_hbm, slot_hbm)

    return k(x_seg, seg_slot.reshape(1, A * _SEG))


def _router_dispatch_body(x_ref, wr_ref, ss_ref, cs_ref, cw_ref):
    xb = x_ref[...].astype(jnp.bfloat16)
    wr = wr_ref[...].astype(jnp.bfloat16)
    logits = jnp.dot(xb, wr, preferred_element_type=jnp.float32)
    lane = lax.broadcasted_iota(jnp.int32, (S, _LANES), 1)
    logits = jnp.where(lane < E, logits, _NEG)

    m1 = jnp.max(logits, axis=1, keepdims=True)
    a1 = jnp.min(jnp.where(logits == m1, lane, _LANES), axis=1, keepdims=True)
    l2 = jnp.where(lane == a1, _NEG, logits)
    m2 = jnp.max(l2, axis=1, keepdims=True)
    a2 = jnp.min(jnp.where(l2 == m2, lane, _LANES), axis=1, keepdims=True)
    t_ = jnp.exp(m2 - m1)
    w1 = 1.0 / (1.0 + t_)
    w2 = t_ / (1.0 + t_)

    sel_mask = (lane == a1) | (lane == a2)
    mf = jnp.where(sel_mask, 1.0, 0.0)

    r_i = lax.broadcasted_iota(jnp.int32, (_LANES, _LANES), 0)
    c_i = lax.broadcasted_iota(jnp.int32, (_LANES, _LANES), 1)
    tl = (c_i <= r_i).astype(jnp.bfloat16)
    off = jnp.zeros((1, _LANES), jnp.float32)
    blocks = []
    for b in range(S // _LANES):
        mb = mf[b * _LANES:(b + 1) * _LANES, :]
        incl = jnp.dot(tl, mb.astype(jnp.bfloat16), preferred_element_type=jnp.float32)
        blocks.append(incl - mb + off)
        off = off + incl[_LANES - 1:_LANES, :]
    c = jnp.concatenate(blocks, axis=0).astype(jnp.int32)

    slot = lane * CAP + c
    kept = c < CAP
    scat = jnp.where(kept, slot, E * CAP)
    comb = jnp.where(kept, slot, lane * CAP)

    def sel(arr, a):
        return jnp.sum(jnp.where(lane == a, arr, 0), axis=1, keepdims=True)

    s1, s2 = sel(scat, a1), sel(scat, a2)
    c1, c2 = sel(comb, a1), sel(comb, a2)
    ss_ref[...] = jnp.concatenate(
        [s1 * _SEG + p for p in range(_SEG)] + [s2 * _SEG + p for p in range(_SEG)],
        axis=1)
    cs_ref[...] = jnp.concatenate(
        [c1 * _SEG + p for p in range(_SEG)] + [c2 * _SEG + p for p in range(_SEG)],
        axis=1)
    k1 = jnp.where(sel(jnp.where(kept, 1, 0), a1) > 0, w1, 0.0)
    k2 = jnp.where(sel(jnp.where(kept, 1, 0), a2) > 0, w2, 0.0)
    cw_ref[...] = jnp.concatenate([k1, k2], axis=1)


def _router_dispatch(x, wr_pad):
    return pl.pallas_call(
        _router_dispatch_body,
        out_shape=[
            jax.ShapeDtypeStruct((S, 2 * _SEG), jnp.int32),
            jax.ShapeDtypeStruct((S, 2 * _SEG), jnp.int32),
            jax.ShapeDtypeStruct((S, 2), jnp.float32),
        ],
    )(x, wr_pad)


def _ffn_body(xe_ref, w1_ref, b1_ref, w2_ref, b2_ref, he_ref):
    xb = xe_ref[...].astype(jnp.bfloat16)
    h = jnp.dot(xb, w1_ref[0], preferred_element_type=jnp.float32)
    h = jax.nn.gelu(h + b1_ref[0], approximate=True)
    o = jnp.dot(h.astype(jnp.bfloat16), w2_ref[0], preferred_element_type=jnp.float32)
    he_ref[...] = o + b2_ref[0]


def _ffn(xe, w1b, b1r, w2b, b2r):
    return pl.pallas_call(
        _ffn_body,
        grid=(E,),
        in_specs=[
            pl.BlockSpec((CAP, D), lambda e: (e, 0)),
            pl.BlockSpec((1, D, F), lambda e: (e, 0, 0)),
            pl.BlockSpec((1, 1, F), lambda e: (e, 0, 0)),
            pl.BlockSpec((1, F, D), lambda e: (e, 0, 0)),
            pl.BlockSpec((1, 1, D), lambda e: (e, 0, 0)),
        ],
        out_specs=pl.BlockSpec((CAP, D), lambda e: (e, 0)),
        out_shape=jax.ShapeDtypeStruct((E * CAP, D), jnp.float32),
    )(xe, w1b, b1r, w2b, b2r)


def _lm_body(hab_ref, cwa_ref, cwb_ref, g_ref, b_ref, wlm_ref, out_ref, hn_ref):
    v = pl.program_id(0)

    @pl.when(v == 0)
    def _():
        comb = cwa_ref[...] * hab_ref[0:S, :] + cwb_ref[...] * hab_ref[S:A, :]
        mu = jnp.mean(comb, axis=1, keepdims=True)
        dd = comb - mu
        var = jnp.mean(dd * dd, axis=1, keepdims=True)
        hn = dd / jnp.sqrt(var + EPS) * g_ref[...] + b_ref[...]
        hn_ref[...] = hn.astype(jnp.bfloat16)

    out_ref[...] = jnp.dot(hn_ref[...], wlm_ref[...], preferred_element_type=jnp.float32)


def _lm_head(hab, cwa, cwb, g, b, wlmb, vblk=1024):
    return pl.pallas_call(
        _lm_body,
        grid=(V // vblk,),
        in_specs=[
            pl.BlockSpec((A, D), lambda v: (0, 0)),
            pl.BlockSpec((S, 1), lambda v: (0, 0)),
            pl.BlockSpec((S, 1), lambda v: (0, 0)),
            pl.BlockSpec((1, D), lambda v: (0, 0)),
            pl.BlockSpec((1, D), lambda v: (0, 0)),
            pl.BlockSpec((D, vblk), lambda v: (0, v)),
        ],
        out_specs=pl.BlockSpec((S, vblk), lambda v: (0, v)),
        out_shape=jax.ShapeDtypeStruct((S, V), jnp.float32),
        scratch_shapes=[pltpu.VMEM((S, D), jnp.bfloat16)],
    )(hab, cwa, cwb, g, b, wlmb)


def kernel(input_ids, embed, w_router, w1, b1, w2, b2, ln_scale, ln_bias, w_lm):
    ids = input_ids.reshape(S).astype(jnp.int32)
    ids_seg = (ids[:, None] * _SEG + jnp.arange(_SEG, dtype=jnp.int32)).reshape(-1)
    x_seg = _sc_gather_rows(embed.reshape(V * _SEG, _WIN), ids_seg)
    x = x_seg.reshape(S, D)

    wr_pad = jnp.zeros((D, _LANES), jnp.float32).at[:, :E].set(w_router)
    ss8, cs8, cw = _router_dispatch(x, wr_pad)

    seg_scat = ss8.reshape(S, 2, _SEG).transpose(1, 0, 2).reshape(A * _SEG)
    seg_comb = cs8.reshape(S, 2, _SEG).transpose(1, 0, 2).reshape(A * _SEG)

    xe_seg = _sc_dispatch_scatter(x_seg, seg_scat)
    he = _ffn(
        xe_seg.reshape(XE_ROWS, D),
        w1.astype(jnp.bfloat16),
        b1.reshape(E, 1, F),
        w2.astype(jnp.bfloat16),
        b2.reshape(E, 1, D),
    )

    hab = _sc_gather_rows(he.reshape(E * CAP * _SEG, _WIN), seg_comb).reshape(A, D)
    logits = _lm_head(
        hab,
        cw[:, 0:1],
        cw[:, 1:2],
        ln_scale.reshape(1, D),
        ln_bias.reshape(1, D),
        w_lm.astype(jnp.bfloat16),
    )
    return logits.reshape(1, S, V)


# speedup vs baseline: 1.5921x; 1.1992x over previous
import functools

import jax
import jax.numpy as jnp
from jax import lax
from jax.experimental import pallas as pl
from jax.experimental.pallas import tpu as pltpu
from jax.experimental.pallas import tpu_sc as plsc

S = 2048
D = 1024
F = 2048
V = 8192
E = 8
CAP = 640
A = 2 * S
XE_ROWS = (E + 1) * CAP
EPS = 1e-5
_LANES = 128
_NEG = -1.0e30


def _sc_mesh():
    return plsc.VectorSubcoreMesh(core_axis_name="core", subcore_axis_name="subcore")


_SEG = 8
_WIN = 128


def _sc_gather_rows(table_seg, idx_seg):
    m8 = idx_seg.shape[0]

    @functools.partial(
        pl.kernel,
        out_type=jax.ShapeDtypeStruct((m8, _WIN), table_seg.dtype),
        mesh=_sc_mesh(),
    )
    def k(tab_hbm, idx_hbm, out_hbm):
        def body(i_vmem, o_vmem):
            pltpu.sync_copy(tab_hbm.at[i_vmem.at[0]], o_vmem)

        pltpu.emit_pipeline(
            body,
            grid=(m8 // _WIN,),
            in_specs=[pl.BlockSpec((1, _WIN), lambda i: (0, i))],
            out_specs=[pl.BlockSpec((_WIN, _WIN), lambda i: (i, 0))],
            core_axis_name=("core", "subcore"),
            dimension_semantics=(pltpu.PARALLEL,),
        )(idx_hbm, out_hbm)

    return k(table_seg, idx_seg.reshape(1, m8))


def _sc_dispatch_scatter(x_seg, seg_slot):
    @functools.partial(
        pl.kernel,
        out_type=jax.ShapeDtypeStruct((XE_ROWS * _SEG, _WIN), x_seg.dtype),
        mesh=_sc_mesh(),
    )
    def k(x_hbm, slot_hbm, xe_hbm):
        def body(x_vmem, i_vmem):
            pltpu.sync_copy(x_vmem, xe_hbm.at[i_vmem.at[0]])

        nsrc = (S * _SEG) // _WIN
        pltpu.emit_pipeline(
            body,
            grid=((A * _SEG) // _WIN,),
            in_specs=[
                pl.BlockSpec((_WIN, _WIN), lambda i: (i % nsrc, 0)),
                pl.BlockSpec((1, _WIN), lambda i: (0, i)),
            ],
            out_specs=[],
            core_axis_name=("core", "subcore"),
            dimension_semantics=(pltpu.PARALLEL,),
        )(x_hbm, slot_hbm)

    return k(x_seg, seg_slot.reshape(1, A * _SEG))


def _router_dispatch_body(x_ref, wr_ref, ss_ref, cs_ref, cw_ref):
    xb = x_ref[...].astype(jnp.bfloat16)
    wr = wr_ref[...].astype(jnp.bfloat16)
    logits = jnp.dot(xb, wr, preferred_element_type=jnp.float32)
    lane = lax.broadcasted_iota(jnp.int32, (S, _LANES), 1)
    logits = jnp.where(lane < E, logits, _NEG)

    m1 = jnp.max(logits, axis=1, keepdims=True)
    a1 = jnp.min(jnp.where(logits == m1, lane, _LANES), axis=1, keepdims=True)
    l2 = jnp.where(lane == a1, _NEG, logits)
    m2 = jnp.max(l2, axis=1, keepdims=True)
    a2 = jnp.min(jnp.where(l2 == m2, lane, _LANES), axis=1, keepdims=True)
    t_ = jnp.exp(m2 - m1)
    w1 = 1.0 / (1.0 + t_)
    w2 = t_ / (1.0 + t_)

    sel_mask = (lane == a1) | (lane == a2)
    mf = jnp.where(sel_mask, 1.0, 0.0)

    r_i = lax.broadcasted_iota(jnp.int32, (_LANES, _LANES), 0)
    c_i = lax.broadcasted_iota(jnp.int32, (_LANES, _LANES), 1)
    tl = (c_i <= r_i).astype(jnp.bfloat16)
    off = jnp.zeros((1, _LANES), jnp.float32)
    blocks = []
    for b in range(S // _LANES):
        mb = mf[b * _LANES:(b + 1) * _LANES, :]
        incl = jnp.dot(tl, mb.astype(jnp.bfloat16), preferred_element_type=jnp.float32)
        blocks.append(incl - mb + off)
        off = off + incl[_LANES - 1:_LANES, :]
    c = jnp.concatenate(blocks, axis=0).astype(jnp.int32)

    slot = lane * CAP + c
    kept = c < CAP
    scat = jnp.where(kept, slot, E * CAP)
    comb = jnp.where(kept, slot, lane * CAP)

    def sel(arr, a):
        return jnp.sum(jnp.where(lane == a, arr, 0), axis=1, keepdims=True)

    s1, s2 = sel(scat, a1), sel(scat, a2)
    c1, c2 = sel(comb, a1), sel(comb, a2)
    ss_ref[...] = jnp.concatenate(
        [s1 * _SEG + p for p in range(_SEG)] + [s2 * _SEG + p for p in range(_SEG)],
        axis=1)
    cs_ref[...] = jnp.concatenate(
        [c1 * _SEG + p for p in range(_SEG)] + [c2 * _SEG + p for p in range(_SEG)],
        axis=1)
    k1 = jnp.where(sel(jnp.where(kept, 1, 0), a1) > 0, w1, 0.0)
    k2 = jnp.where(sel(jnp.where(kept, 1, 0), a2) > 0, w2, 0.0)
    cw_ref[...] = jnp.concatenate([k1, k2], axis=1)


def _router_dispatch(x, wr_pad):
    return pl.pallas_call(
        _router_dispatch_body,
        out_shape=[
            jax.ShapeDtypeStruct((S, 2 * _SEG), jnp.int32),
            jax.ShapeDtypeStruct((S, 2 * _SEG), jnp.int32),
            jax.ShapeDtypeStruct((S, 2), jnp.float32),
        ],
    )(x, wr_pad)


def _ffn_body(xe_ref, w1_ref, b1_ref, w2_ref, b2_ref, he_ref):
    xb = xe_ref[...].astype(jnp.bfloat16)
    h = jnp.dot(xb, w1_ref[0].astype(jnp.bfloat16), preferred_element_type=jnp.float32)
    h = jax.nn.gelu(h + b1_ref[0], approximate=True)
    o = jnp.dot(h.astype(jnp.bfloat16), w2_ref[0].astype(jnp.bfloat16),
                preferred_element_type=jnp.float32)
    he_ref[...] = o + b2_ref[0]


def _ffn(xe, w1b, b1r, w2b, b2r):
    return pl.pallas_call(
        _ffn_body,
        grid=(E,),
        in_specs=[
            pl.BlockSpec((CAP, D), lambda e: (e, 0)),
            pl.BlockSpec((1, D, F), lambda e: (e, 0, 0)),
            pl.BlockSpec((1, 1, F), lambda e: (e, 0, 0)),
            pl.BlockSpec((1, F, D), lambda e: (e, 0, 0)),
            pl.BlockSpec((1, 1, D), lambda e: (e, 0, 0)),
        ],
        out_specs=pl.BlockSpec((CAP, D), lambda e: (e, 0)),
        out_shape=jax.ShapeDtypeStruct((E * CAP, D), jnp.float32),
    )(xe, w1b, b1r, w2b, b2r)


def _lm_body(hab_ref, cwa_ref, cwb_ref, g_ref, b_ref, wlm_ref, out_ref, hn_ref):
    v = pl.program_id(0)

    @pl.when(v == 0)
    def _():
        comb = cwa_ref[...] * hab_ref[0:S, :] + cwb_ref[...] * hab_ref[S:A, :]
        mu = jnp.mean(comb, axis=1, keepdims=True)
        dd = comb - mu
        var = jnp.mean(dd * dd, axis=1, keepdims=True)
        hn = dd / jnp.sqrt(var + EPS) * g_ref[...] + b_ref[...]
        hn_ref[...] = hn.astype(jnp.bfloat16)

    out_ref[...] = jnp.dot(hn_ref[...], wlm_ref[...].astype(jnp.bfloat16),
                           preferred_element_type=jnp.float32)


def _lm_head(hab, cwa, cwb, g, b, wlmb, vblk=512):
    return pl.pallas_call(
        _lm_body,
        grid=(V // vblk,),
        in_specs=[
            pl.BlockSpec((A, D), lambda v: (0, 0)),
            pl.BlockSpec((S, 1), lambda v: (0, 0)),
            pl.BlockSpec((S, 1), lambda v: (0, 0)),
            pl.BlockSpec((1, D), lambda v: (0, 0)),
            pl.BlockSpec((1, D), lambda v: (0, 0)),
            pl.BlockSpec((D, vblk), lambda v: (0, v)),
        ],
        out_specs=pl.BlockSpec((S, vblk), lambda v: (0, v)),
        out_shape=jax.ShapeDtypeStruct((S, V), jnp.float32),
        scratch_shapes=[pltpu.VMEM((S, D), jnp.bfloat16)],
    )(hab, cwa, cwb, g, b, wlmb)


def kernel(input_ids, embed, w_router, w1, b1, w2, b2, ln_scale, ln_bias, w_lm):
    ids = input_ids.reshape(S).astype(jnp.int32)
    ids_seg = (ids[:, None] * _SEG + jnp.arange(_SEG, dtype=jnp.int32)).reshape(-1)
    x_seg = _sc_gather_rows(embed.reshape(V * _SEG, _WIN), ids_seg)
    x = x_seg.reshape(S, D)

    wr_pad = jnp.zeros((D, _LANES), jnp.float32).at[:, :E].set(w_router)
    ss8, cs8, cw = _router_dispatch(x, wr_pad)

    seg_scat = ss8.reshape(S, 2, _SEG).transpose(1, 0, 2).reshape(A * _SEG)
    seg_comb = cs8.reshape(S, 2, _SEG).transpose(1, 0, 2).reshape(A * _SEG)

    xe_seg = _sc_dispatch_scatter(x_seg, seg_scat)
    he = _ffn(
        xe_seg.reshape(XE_ROWS, D),
        w1,
        b1.reshape(E, 1, F),
        w2,
        b2.reshape(E, 1, D),
    )

    hab = _sc_gather_rows(he.reshape(E * CAP * _SEG, _WIN), seg_comb).reshape(A, D)
    logits = _lm_head(
        hab,
        cw[:, 0:1],
        cw[:, 1:2],
        ln_scale.reshape(1, D),
        ln_bias.reshape(1, D),
        w_lm,
    )
    return logits.reshape(1, S, V)
